# baseline (speedup 1.0000x reference)
import functools
import jax
import jax.numpy as jnp
from jax import lax
from jax.experimental import pallas as pl
from jax.experimental.pallas import tpu as pltpu
from jax.experimental.pallas import tpu_sc as plsc

_NC = 2
_NS = 16
_NW = _NC * _NS
_S = 4096 * 1024
_N = 3 * _S
_PW = _S // _NW
_CH = 65536
_NJ = _PW // _CH
_CB = 32768
_NB = _PW // _CB


@functools.partial(
    pl.kernel,
    mesh=plsc.VectorSubcoreMesh(core_axis_name="c", subcore_axis_name="s"),
    out_type=[
        jax.ShapeDtypeStruct((_N,), jnp.float32),
        jax.ShapeDtypeStruct((_N,), jnp.float32),
    ],
    scratch_types=[
        pltpu.VMEM((_CH,), jnp.float32),
        pltpu.VMEM((_CB,), jnp.float32),
    ],
)
def _sc_fused(x_hbm, out1_hbm, out2_hbm, buf, cbuf):
    wid = lax.axis_index("s") * _NC + lax.axis_index("c")

    def _fill(i, carry):
        cbuf[pl.ds(i * 16, 16)] = jnp.full((16,), 2.0, dtype=jnp.float32)
        return carry

    lax.fori_loop(0, _CB // 16, _fill, 0)

    for b in range(_NB):
        off0 = wid * _PW + b * _CB
        pltpu.sync_copy(cbuf, out2_hbm.at[pl.ds(off0, _CB)])

    for s in range(3):
        for j in range(_NJ):
            off = s * _S + wid * _PW + j * _CH
            pltpu.sync_copy(x_hbm.at[pl.ds(off, _CH)], buf)
            pltpu.sync_copy(buf, out1_hbm.at[pl.ds(off, _CH)])
            if s > 0:
                pltpu.sync_copy(buf, out2_hbm.at[pl.ds(off, _CH)])


def kernel(x):
    xf = x.reshape(_N)
    out1, out2 = _sc_fused(xf)
    return (out1.reshape(x.shape), out2.reshape(x.shape))


# speedup vs baseline: 1.0268x; 1.0268x over previous
import functools
import jax
import jax.numpy as jnp
from jax import lax
from jax.experimental import pallas as pl
from jax.experimental.pallas import tpu as pltpu
from jax.experimental.pallas import tpu_sc as plsc

_NC = 2
_NS = 16
_NW = _NC * _NS
_S = 4096 * 1024
_N = 3 * _S
_PW = _S // _NW
_CH = 32768
_NJ = _PW // _CH
_CB = 16384
_NB = _PW // _CB
_NCHUNK = 3 * _NJ


@functools.partial(
    pl.kernel,
    mesh=plsc.VectorSubcoreMesh(core_axis_name="c", subcore_axis_name="s"),
    out_type=[
        jax.ShapeDtypeStruct((_N,), jnp.float32),
        jax.ShapeDtypeStruct((_N,), jnp.float32),
    ],
    scratch_types=[
        pltpu.VMEM((_CH,), jnp.float32),
        pltpu.VMEM((_CH,), jnp.float32),
        pltpu.VMEM((_CB,), jnp.float32),
        pltpu.SemaphoreType.DMA,
        pltpu.SemaphoreType.DMA,
        pltpu.SemaphoreType.DMA,
        pltpu.SemaphoreType.DMA,
        pltpu.SemaphoreType.DMA,
    ],
)
def _sc_fused(x_hbm, out1_hbm, out2_hbm, b0, b1, cbuf, sr0, sr1, sw0, sw1, swc):
    wid = lax.axis_index("s") * _NC + lax.axis_index("c")
    bufs = (b0, b1)
    srs = (sr0, sr1)
    sws = (sw0, sw1)

    def _fill(i, carry):
        cbuf[pl.ds(i * 16, 16)] = jnp.full((16,), 2.0, dtype=jnp.float32)
        return carry

    lax.fori_loop(0, _CB // 16, _fill, 0)

    memset_hs = []
    for b in range(_NB):
        off0 = wid * _PW + b * _CB
        memset_hs.append(
            pltpu.async_copy(cbuf, out2_hbm.at[pl.ds(off0, _CB)], swc))

    def _off(k):
        s, j = divmod(k, _NJ)
        return s * _S + wid * _PW + j * _CH

    read_hs = {}
    wpend = [[], []]
    read_hs[0] = pltpu.async_copy(x_hbm.at[pl.ds(_off(0), _CH)], bufs[0], srs[0])
    for k in range(_NCHUNK):
        cur = k & 1
        nxt = (k + 1) & 1
        if k + 1 < _NCHUNK:
            for h in wpend[nxt]:
                h.wait()
            wpend[nxt] = []
            read_hs[k + 1] = pltpu.async_copy(
                x_hbm.at[pl.ds(_off(k + 1), _CH)], bufs[nxt], srs[nxt])
        read_hs[k].wait()
        dst = pl.ds(_off(k), _CH)
        wpend[cur].append(pltpu.async_copy(bufs[cur], out1_hbm.at[dst], sws[cur]))
        if k >= _NJ:
            wpend[cur].append(pltpu.async_copy(bufs[cur], out2_hbm.at[dst], sws[cur]))
    for lst in wpend:
        for h in lst:
            h.wait()
    for h in memset_hs:
        h.wait()


def kernel(x):
    xf = x.reshape(_N)
    out1, out2 = _sc_fused(xf)
    return (out1.reshape(x.shape), out2.reshape(x.shape))


# speedup vs baseline: 4.7799x; 4.6553x over previous
import jax
import jax.numpy as jnp
from jax.experimental import pallas as pl


_BR = 512


def _body(x_ref, o1_ref, o2_ref):
    v = x_ref[...]
    o1_ref[...] = v
    lead = jax.lax.broadcasted_iota(jnp.int32, v.shape, 0)
    o2_ref[...] = jnp.where(lead == 0, jnp.float32(2.0), v)


def kernel(x):
    n, r, c = x.shape
    grid = (r // _BR,)
    spec = pl.BlockSpec((n, _BR, c), lambda i: (0, i, 0))
    out1, out2 = pl.pallas_call(
        _body,
        grid=grid,
        in_specs=[spec],
        out_specs=[spec, spec],
        out_shape=[
            jax.ShapeDtypeStruct(x.shape, x.dtype),
            jax.ShapeDtypeStruct(x.shape, x.dtype),
        ],
    )(x)
    return (out1, out2)
